# baseline (speedup 1.0000x reference)
import math

import jax
import jax.numpy as jnp
from jax import lax
from jax.experimental import pallas as pl
from jax.experimental.pallas import tpu as pltpu
from jax.experimental.pallas import tpu_sc as plsc

_NUM_EMB = 1_000_000
_HALF = _NUM_EMB // 2
_DIM = 32
_K = 8
_ROWS = 16384
_COLS = 26
_N = _ROWS * _COLS
_NC = 2
_NS = 16
_NW = _NC * _NS
_PER_W = _N // _NW
_E = 128
_UNITS = _PER_W // _E
_LANES = 16
_NBUF = 2
_SCALE = 1.0 / math.sqrt(_K)


def _body(table_hbm, ids_hbm, out_hbm, ids_all, idx_vs, gbufs, stg, sem_g,
          sem_o):
    wid = lax.axis_index("s") * _NC + lax.axis_index("c")
    wbase = wid * jnp.int32(_PER_W)
    pltpu.sync_copy(ids_hbm.at[pl.ds(wbase, _PER_W)], ids_all)

    def compute_idx(u, b):
        off = u * jnp.int32(_E)

        def grp(i, c):
            i16 = i * jnp.int32(_LANES)
            x = ids_all[pl.ds(off + i16, _LANES)]
            idx_vs[b][0][pl.ds(i16, _LANES)] = x
            r = x
            for j in range(1, _K):
                r2 = r + r
                r = jnp.where(r >= _HALF, r2 - _NUM_EMB, r2)
                idx_vs[b][j][pl.ds(i16, _LANES)] = r
            return c

        lax.fori_loop(jnp.int32(0), jnp.int32(_E // _LANES), grp, jnp.int32(0))

    def reduce_scale(b):
        def s(i, c):
            row = i * jnp.int32(4)
            for rr in range(4):
                for h in (0, _LANES):
                    v = gbufs[b][0][row + rr, pl.ds(h, _LANES)]
                    for j in range(1, _K):
                        v = v + gbufs[b][j][row + rr, pl.ds(h, _LANES)]
                    stg[b][row + rr, pl.ds(h, _LANES)] = v * _SCALE
            return c

        lax.fori_loop(jnp.int32(0), jnp.int32(_E // 4), s, jnp.int32(0))

    def wait_out(b):
        pltpu.make_async_copy(
            stg[b], out_hbm.at[pl.ds(0, _E)], sem_o[b]).wait()

    def issue(u, b):
        compute_idx(u, b)
        for j in range(_K):
            pltpu.async_copy(
                table_hbm.at[idx_vs[b][j]], gbufs[b][j], sem_g[b])

    def complete(u, b, wait_mode):
        for j in range(_K):
            pltpu.make_async_copy(
                table_hbm.at[idx_vs[b][j]], gbufs[b][j], sem_g[b]).wait()
        if wait_mode == "always":
            wait_out(b)
        elif wait_mode == "guard":
            @pl.when(u >= jnp.int32(_NBUF))
            def _():
                wait_out(b)
        reduce_scale(b)
        pltpu.async_copy(
            stg[b], out_hbm.at[pl.ds(wbase + u * jnp.int32(_E), _E)],
            sem_o[b])

    issue(jnp.int32(0), 0)

    def grp(g, carry):
        u0 = g * jnp.int32(2)
        issue(u0 + 1, 1)
        complete(u0, 0, "guard")
        issue(u0 + 2, 0)
        complete(u0 + 1, 1, "guard")
        return carry

    n_grps = (_UNITS - 2) // 2
    lax.fori_loop(jnp.int32(0), jnp.int32(n_grps), grp, jnp.int32(0))

    issue(jnp.int32(_UNITS - 1), 1)
    complete(jnp.int32(_UNITS - 2), 0, "always")
    complete(jnp.int32(_UNITS - 1), 1, "always")
    for b in range(_NBUF):
        wait_out(b)


def kernel(id_, emb_weight):
    ids = id_.reshape(_N).astype(jnp.int32)
    mesh = plsc.VectorSubcoreMesh(
        core_axis_name="c", subcore_axis_name="s",
        num_cores=_NC, num_subcores=_NS)
    out = pl.kernel(
        _body,
        out_type=jax.ShapeDtypeStruct((_N, _DIM), jnp.float32),
        mesh=mesh,
        compiler_params=pltpu.CompilerParams(use_tc_tiling_on_sc=False),
        scratch_types=[
            pltpu.VMEM((_PER_W,), jnp.int32),
            [[pltpu.VMEM((_E,), jnp.int32) for _ in range(_K)]
             for _ in range(_NBUF)],
            [[pltpu.VMEM((_E, _DIM), jnp.float32) for _ in range(_K)]
             for _ in range(_NBUF)],
            [pltpu.VMEM((_E, _DIM), jnp.float32) for _ in range(_NBUF)],
            [pltpu.SemaphoreType.DMA for _ in range(_NBUF)],
            [pltpu.SemaphoreType.DMA for _ in range(_NBUF)],
        ],
    )(emb_weight, ids)
    return out.reshape(_ROWS, _COLS, _DIM)
